# baseline (speedup 1.0000x reference)
import functools

import jax
import jax.numpy as jnp
from jax import lax
from jax.experimental import pallas as pl
from jax.experimental.pallas import tpu as pltpu
from jax.experimental.pallas import tpu_sc as plsc

N_NODES = 10000
N_EDGES = 160000
D_IN = 256
D_OUT = 256

NC = 2
NS = 16
K = 128
NWIN = 80
E_PAD = NS * NWIN * K
EDGES_PER_SUBCORE = NWIN * K
HALF = 128
N_PAD = 10240
ROWS_PER_SUBCORE = N_PAD // NS


def _sc_agg(packed2, norm2, h2):
    mesh = plsc.VectorSubcoreMesh(core_axis_name="c", subcore_axis_name="s")

    @functools.partial(
        pl.kernel,
        out_type=jax.ShapeDtypeStruct((NC, N_NODES, HALF), jnp.float32),
        mesh=mesh,
        scratch_types=[
            pltpu.VMEM((NWIN, K), jnp.int32),
            pltpu.VMEM((2, K), jnp.int32),
            pltpu.VMEM((2, K), jnp.int32),
            pltpu.VMEM((2, K), jnp.float32),
            pltpu.VMEM((2, K, HALF), jnp.float32),
            pltpu.VMEM_SHARED((N_PAD, HALF), jnp.float32),
            pltpu.SemaphoreType.DMA,
            pltpu.SemaphoreType.DMA,
            pltpu.SemaphoreType.DMA,
            pltpu.SemaphoreType.DMA,
        ],
    )
    def sc_kernel(packed_hbm, norm_hbm, h2_hbm, out_hbm,
                  pk_v, src_w, dst_w, norm_w, rows_v, acc,
                  gsem0, gsem1, nsem0, nsem1):
        c = lax.axis_index("c")
        s = lax.axis_index("s")
        base = s * NWIN

        pltpu.sync_copy(packed_hbm.at[pl.ds(base, NWIN)], pk_v)

        @pl.loop(0, K)
        def _(r):
            for j in range(HALF // 16):
                rows_v[0, r, pl.ds(j * 16, 16)] = jnp.zeros((16,), jnp.float32)

        row0 = s * ROWS_PER_SUBCORE
        for i in range(ROWS_PER_SUBCORE // K):
            pltpu.sync_copy(rows_v.at[0],
                            acc.at[pl.ds(row0 + i * K, K)])

        plsc.subcore_barrier()

        gsems = (gsem0, gsem1)
        nsems = (nsem0, nsem1)

        def unpack(w, p):
            for j in range(K // 16):
                sl = pl.ds(j * 16, 16)
                v = pk_v[w, sl]
                src_w[p, sl] = ((v & 0xFFFF) << 1) + c
                dst_w[p, sl] = v >> 16

        def gather(p):
            return pltpu.make_async_copy(h2_hbm.at[src_w.at[p]],
                                         rows_v.at[p], gsems[p])

        def normcp(w, p):
            return pltpu.make_async_copy(norm_hbm.at[base + w],
                                         norm_w.at[p], nsems[p])

        def prefetch(w, p):
            unpack(w, p)
            gather(p).start()
            normcp(w, p).start()

        def scale_and_scatter(p):
            @pl.loop(0, K, step=16)
            def _(g):
                nv = norm_w[p, pl.ds(g, 16)]
                for i in range(16):
                    t = nv[i]
                    vals = [rows_v[p, g + i, pl.ds(j * 16, 16)]
                            for j in range(HALF // 16)]
                    for j in range(HALF // 16):
                        rows_v[p, g + i, pl.ds(j * 16, 16)] = vals[j] * t

            pltpu.sync_copy(rows_v.at[p], acc.at[dst_w.at[p]], add=True)

        prefetch(0, 0)
        prefetch(1, 1)

        @pl.loop(0, NWIN, step=2)
        def _(w):
            for p in range(2):
                gather(p).wait()
                normcp(w + p, p).wait()
                scale_and_scatter(p)

                @pl.when(w + 2 + p < NWIN)
                def _():
                    prefetch(w + 2 + p, p)

        plsc.subcore_barrier()

        last_rows = N_NODES - (NS - 1) * ROWS_PER_SUBCORE

        @pl.when(s < NS - 1)
        def _():
            pltpu.sync_copy(acc.at[pl.ds(row0, ROWS_PER_SUBCORE)],
                            out_hbm.at[c].at[pl.ds(row0, ROWS_PER_SUBCORE)])

        @pl.when(s == NS - 1)
        def _():
            pltpu.sync_copy(acc.at[pl.ds(row0, last_rows)],
                            out_hbm.at[c].at[pl.ds(row0, last_rows)])

    return sc_kernel(packed2, norm2, h2)


def _tc_matmul_body(a0_ref, a1_ref, w_ref, b_ref, o_ref):
    dn = (((1,), (1,)), ((), ()))
    acc = lax.dot_general(a0_ref[...], w_ref[:, 0:HALF], dn,
                          preferred_element_type=jnp.float32)
    acc = acc + lax.dot_general(a1_ref[...], w_ref[:, HALF:D_IN], dn,
                                preferred_element_type=jnp.float32)
    o_ref[...] = acc + b_ref[...]


def _tc_matmul(a0, a1, W, b2d):
    blk = 1000
    grid = (N_NODES // blk,)
    return pl.pallas_call(
        _tc_matmul_body,
        grid=grid,
        in_specs=[
            pl.BlockSpec((blk, HALF), lambda i: (i, 0)),
            pl.BlockSpec((blk, HALF), lambda i: (i, 0)),
            pl.BlockSpec((D_OUT, D_IN), lambda i: (0, 0)),
            pl.BlockSpec((1, D_OUT), lambda i: (0, 0)),
        ],
        out_specs=pl.BlockSpec((blk, D_OUT), lambda i: (i, 0)),
        out_shape=jax.ShapeDtypeStruct((N_NODES, D_OUT), jnp.float32),
    )(a0, a1, W, b2d)


def kernel(h, edge_index, edge_norm, W, b):
    src = edge_index[0].astype(jnp.int32)
    dst = edge_index[1].astype(jnp.int32)
    norm = edge_norm.reshape(-1).astype(jnp.float32)

    pad = E_PAD - N_EDGES
    fill = (jnp.arange(pad, dtype=jnp.int32) * 7919) % N_NODES
    src_p = jnp.concatenate([src, fill])
    dst_p = jnp.concatenate([dst, fill])
    norm_p = jnp.concatenate([norm, jnp.zeros((pad,), jnp.float32)])

    packed2 = (src_p | (dst_p << 16)).reshape(NS * NWIN, K)
    norm2 = norm_p.reshape(NS * NWIN, K)
    h2 = h.reshape(2 * N_NODES, HALF)

    acc = _sc_agg(packed2, norm2, h2)
    return _tc_matmul(acc[0], acc[1], W, b.reshape(1, D_OUT))


# speedup vs baseline: 1.0476x; 1.0476x over previous
import dataclasses
import functools

import jax
import jax.numpy as jnp
from jax import lax
from jax.experimental import pallas as pl
from jax.experimental.pallas import tpu as pltpu
from jax.experimental.pallas import tpu_sc as plsc

N_NODES = 10000
N_EDGES = 160000
D_IN = 256
D_OUT = 256

NC = 2
NS = 16
K = 80
NWIN = 128
E_PAD = NS * NWIN * K
HALF = 128
N_PAD = 10240
ROWS_PER_SUBCORE = N_PAD // NS
RBUF = 4
MBUF = 8
GDIST = 3


def _sc_compiler_params():
    cp = pltpu.CompilerParams()
    if "needs_layout_passes" in pltpu.CompilerParams.__dataclass_fields__:
        cp = dataclasses.replace(cp, needs_layout_passes=False)
    return cp


def _sc_agg(meta3, h2):
    mesh = plsc.VectorSubcoreMesh(core_axis_name="c", subcore_axis_name="s")

    @functools.partial(
        pl.kernel,
        out_type=jax.ShapeDtypeStruct((NC, N_NODES, HALF), jnp.float32),
        mesh=mesh,
        scratch_types=[
            pltpu.VMEM((MBUF, 2, K), jnp.int32),
            pltpu.VMEM((MBUF, K), jnp.int32),
            pltpu.VMEM((MBUF, K), jnp.int32),
            pltpu.VMEM((RBUF, K, HALF), jnp.float32),
            pltpu.VMEM_SHARED((N_PAD, HALF), jnp.float32),
        ]
        + [pltpu.SemaphoreType.DMA] * (MBUF + 2 * RBUF + 1),
        compiler_params=_sc_compiler_params(),
    )
    def sc_kernel(meta_hbm, h2_hbm, out_hbm,
                  meta_w, idx_w, dst_w, rows_v, acc, *sems):
        msems = sems[:MBUF]
        gsems = sems[MBUF:MBUF + RBUF]
        ssems = sems[MBUF + RBUF:MBUF + 2 * RBUF]
        zsem = sems[MBUF + 2 * RBUF]
        c = lax.axis_index("c")
        s = lax.axis_index("s")
        base = s * NWIN

        @pl.loop(0, K)
        def _(r):
            for j in range(HALF // 16):
                rows_v[0, r, pl.ds(j * 16, 16)] = jnp.zeros((16,), jnp.float32)

        row0 = s * ROWS_PER_SUBCORE
        nz = ROWS_PER_SUBCORE // K
        zcopies = [pltpu.make_async_copy(rows_v.at[0],
                                         acc.at[pl.ds(row0 + i * K, K)], zsem)
                   for i in range(nz)]
        for z in zcopies:
            z.start()
        for z in zcopies:
            z.wait()

        plsc.subcore_barrier()

        def meta_cp(w, m):
            return pltpu.make_async_copy(meta_hbm.at[base + w],
                                         meta_w.at[m], msems[m])

        def unpack(m):
            for j in range(K // 16):
                sl = pl.ds(j * 16, 16)
                v = meta_w[m, 0, sl]
                idx_w[m, sl] = ((v & 0xFFFF) << 1) + c
                dst_w[m, sl] = v >> 16

        def gather(r, m):
            return pltpu.make_async_copy(h2_hbm.at[idx_w.at[m]],
                                         rows_v.at[r], gsems[r])

        def scatter_start(r, m):
            pltpu.async_copy(rows_v.at[r], acc.at[dst_w.at[m]],
                             ssems[r], add=True)

        def scatter_wait(r, m):
            pltpu.make_async_copy(rows_v.at[r], acc.at[dst_w.at[m]],
                                  ssems[r]).wait()

        def scale(r, m):
            @pl.loop(0, K, step=16)
            def _(g):
                nv = plsc.bitcast(meta_w[m, 1, pl.ds(g, 16)], jnp.float32)
                for i in range(16):
                    t = nv[i]
                    vals = [rows_v[r, g + i, pl.ds(j * 16, 16)]
                            for j in range(HALF // 16)]
                    for j in range(HALF // 16):
                        rows_v[r, g + i, pl.ds(j * 16, 16)] = vals[j] * t

        def step(wp, q, first):
            r = q % RBUF
            m = q % MBUF
            gather(r, m).wait()
            scale(r, m)
            scatter_start(r, m)
            if first:
                if wp + MBUF < NWIN:
                    meta_cp(wp + MBUF, m).start()
                if wp + GDIST < NWIN:
                    m3 = (q + GDIST) % MBUF
                    r3 = (q + GDIST) % RBUF
                    meta_cp(wp + GDIST, m3).wait()
                    unpack(m3)
                    if wp + GDIST >= RBUF:
                        scatter_wait(r3, (q + GDIST - RBUF) % MBUF)
                    gather(r3, m3).start()
            else:
                @pl.when(wp + MBUF < NWIN)
                def _():
                    meta_cp(wp + MBUF, m).start()

                @pl.when(wp + GDIST < NWIN)
                def _():
                    m3 = (q + GDIST) % MBUF
                    r3 = (q + GDIST) % RBUF
                    meta_cp(wp + GDIST, m3).wait()
                    unpack(m3)
                    scatter_wait(r3, (q + GDIST - RBUF) % MBUF)
                    gather(r3, m3).start()

        for m in range(MBUF):
            meta_cp(m, m).start()
        for q in range(GDIST):
            meta_cp(q, q).wait()
            unpack(q)
            gather(q % RBUF, q).start()
        for wp in range(MBUF):
            step(wp, wp, first=True)

        @pl.loop(MBUF, NWIN, step=MBUF)
        def _(w):
            for q in range(MBUF):
                step(w + q, q, first=False)

        for wp in range(NWIN - GDIST, NWIN):
            scatter_wait(wp % RBUF, wp % MBUF)

        plsc.subcore_barrier()

        last_rows = N_NODES - (NS - 1) * ROWS_PER_SUBCORE

        @pl.when(s < NS - 1)
        def _():
            pltpu.sync_copy(acc.at[pl.ds(row0, ROWS_PER_SUBCORE)],
                            out_hbm.at[c].at[pl.ds(row0, ROWS_PER_SUBCORE)])

        @pl.when(s == NS - 1)
        def _():
            pltpu.sync_copy(acc.at[pl.ds(row0, last_rows)],
                            out_hbm.at[c].at[pl.ds(row0, last_rows)])

    return sc_kernel(meta3, h2)


def _tc_matmul_body(a0_ref, a1_ref, w_ref, b_ref, o_ref):
    dn = (((1,), (1,)), ((), ()))
    acc = lax.dot_general(a0_ref[...], w_ref[:, 0:HALF], dn,
                          preferred_element_type=jnp.float32)
    acc = acc + lax.dot_general(a1_ref[...], w_ref[:, HALF:D_IN], dn,
                                preferred_element_type=jnp.float32)
    o_ref[...] = acc + b_ref[...]


def _tc_matmul(a0, a1, W, b2d):
    blk = 1000
    grid = (N_NODES // blk,)
    return pl.pallas_call(
        _tc_matmul_body,
        grid=grid,
        in_specs=[
            pl.BlockSpec((blk, HALF), lambda i: (i, 0)),
            pl.BlockSpec((blk, HALF), lambda i: (i, 0)),
            pl.BlockSpec((D_OUT, D_IN), lambda i: (0, 0)),
            pl.BlockSpec((1, D_OUT), lambda i: (0, 0)),
        ],
        out_specs=pl.BlockSpec((blk, D_OUT), lambda i: (i, 0)),
        out_shape=jax.ShapeDtypeStruct((N_NODES, D_OUT), jnp.float32),
    )(a0, a1, W, b2d)


def kernel(h, edge_index, edge_norm, W, b):
    src = edge_index[0].astype(jnp.int32)
    dst = edge_index[1].astype(jnp.int32)
    norm = edge_norm.reshape(-1).astype(jnp.float32)

    pad = E_PAD - N_EDGES
    fill = (jnp.arange(pad, dtype=jnp.int32) * 7919) % N_NODES
    src_p = jnp.concatenate([src, fill])
    dst_p = jnp.concatenate([dst, fill])
    norm_p = jnp.concatenate([norm, jnp.zeros((pad,), jnp.float32)])

    packed2 = (src_p | (dst_p << 16)).reshape(NS * NWIN, K)
    nbits2 = norm_p.view(jnp.int32).reshape(NS * NWIN, K)
    meta3 = jnp.stack([packed2, nbits2], axis=1)
    h2 = h.reshape(2 * N_NODES, HALF)

    acc = _sc_agg(meta3, h2)
    return _tc_matmul(acc[0], acc[1], W, b.reshape(1, D_OUT))


# speedup vs baseline: 1.1418x; 1.0899x over previous
import dataclasses
import functools

import jax
import jax.numpy as jnp
from jax import lax
from jax.experimental import pallas as pl
from jax.experimental.pallas import tpu as pltpu
from jax.experimental.pallas import tpu_sc as plsc

N_NODES = 10000
N_EDGES = 160000
D_IN = 256
D_OUT = 256

NC = 2
NS = 16
K = 80
NWIN = 128
E_PAD = NS * NWIN * K
HALF = 128
N_PAD = 10240
ROWS_PER_SUBCORE = N_PAD // NS
RBUF = 4
MBUF = 8
GDIST = 3


def _sc_compiler_params():
    cp = pltpu.CompilerParams()
    if "needs_layout_passes" in pltpu.CompilerParams.__dataclass_fields__:
        cp = dataclasses.replace(cp, needs_layout_passes=False)
    return cp


def _sc_agg(meta3, hab, b2):
    mesh = plsc.VectorSubcoreMesh(core_axis_name="c", subcore_axis_name="s")

    @functools.partial(
        pl.kernel,
        out_type=jax.ShapeDtypeStruct((N_NODES, D_OUT), jnp.float32),
        mesh=mesh,
        scratch_types=[
            pltpu.VMEM((MBUF, 2, K), jnp.int32),
            pltpu.VMEM((MBUF, K), jnp.int32),
            pltpu.VMEM((MBUF, K), jnp.int32),
            pltpu.VMEM((RBUF, K, HALF), jnp.float32),
            pltpu.VMEM((HALF,), jnp.float32),
            pltpu.VMEM_SHARED((N_PAD, HALF), jnp.float32),
        ]
        + [pltpu.SemaphoreType.DMA] * (MBUF + 2 * RBUF + 1),
        compiler_params=_sc_compiler_params(),
    )
    def sc_kernel(meta_hbm, hab_hbm, b_hbm, out_hbm,
                  meta_w, idx_w, dst_w, rows_v, b_v, acc, *sems):
        msems = sems[:MBUF]
        gsems = sems[MBUF:MBUF + RBUF]
        ssems = sems[MBUF + RBUF:MBUF + 2 * RBUF]
        zsem = sems[MBUF + 2 * RBUF]
        c = lax.axis_index("c")
        s = lax.axis_index("s")
        base = s * NWIN

        pltpu.sync_copy(b_hbm.at[c], b_v)

        @pl.loop(0, K)
        def _(r):
            for j in range(HALF // 16):
                rows_v[0, r, pl.ds(j * 16, 16)] = b_v[pl.ds(j * 16, 16)]

        row0 = s * ROWS_PER_SUBCORE
        nz = ROWS_PER_SUBCORE // K
        zcopies = [pltpu.make_async_copy(rows_v.at[0],
                                         acc.at[pl.ds(row0 + i * K, K)], zsem)
                   for i in range(nz)]
        for z in zcopies:
            z.start()
        for z in zcopies:
            z.wait()

        plsc.subcore_barrier()

        def meta_cp(w, m):
            return pltpu.make_async_copy(meta_hbm.at[base + w],
                                         meta_w.at[m], msems[m])

        def unpack(m):
            for j in range(K // 16):
                sl = pl.ds(j * 16, 16)
                v = meta_w[m, 0, sl]
                idx_w[m, sl] = v & 0xFFFF
                dst_w[m, sl] = v >> 16

        def gather(r, m):
            return pltpu.make_async_copy(hab_hbm.at[c].at[idx_w.at[m]],
                                         rows_v.at[r], gsems[r])

        def scatter_start(r, m):
            pltpu.async_copy(rows_v.at[r], acc.at[dst_w.at[m]],
                             ssems[r], add=True)

        def scatter_wait(r, m):
            pltpu.make_async_copy(rows_v.at[r], acc.at[dst_w.at[m]],
                                  ssems[r]).wait()

        def scale(r, m):
            @pl.loop(0, K, step=16)
            def _(g):
                nv = plsc.bitcast(meta_w[m, 1, pl.ds(g, 16)], jnp.float32)
                for i in range(16):
                    t = nv[i]
                    vals = [rows_v[r, g + i, pl.ds(j * 16, 16)]
                            for j in range(HALF // 16)]
                    for j in range(HALF // 16):
                        rows_v[r, g + i, pl.ds(j * 16, 16)] = vals[j] * t

        def step(wp, q, first):
            r = q % RBUF
            m = q % MBUF
            gather(r, m).wait()
            scale(r, m)
            scatter_start(r, m)
            if first:
                if wp + MBUF < NWIN:
                    meta_cp(wp + MBUF, m).start()
                if wp + GDIST < NWIN:
                    m3 = (q + GDIST) % MBUF
                    r3 = (q + GDIST) % RBUF
                    meta_cp(wp + GDIST, m3).wait()
                    unpack(m3)
                    if wp + GDIST >= RBUF:
                        scatter_wait(r3, (q + GDIST - RBUF) % MBUF)
                    gather(r3, m3).start()
            else:
                @pl.when(wp + MBUF < NWIN)
                def _():
                    meta_cp(wp + MBUF, m).start()

                @pl.when(wp + GDIST < NWIN)
                def _():
                    m3 = (q + GDIST) % MBUF
                    r3 = (q + GDIST) % RBUF
                    meta_cp(wp + GDIST, m3).wait()
                    unpack(m3)
                    scatter_wait(r3, (q + GDIST - RBUF) % MBUF)
                    gather(r3, m3).start()

        for m in range(MBUF):
            meta_cp(m, m).start()
        for q in range(GDIST):
            meta_cp(q, q).wait()
            unpack(q)
            gather(q % RBUF, q).start()
        for wp in range(MBUF):
            step(wp, wp, first=True)

        @pl.loop(MBUF, NWIN, step=MBUF)
        def _(w):
            for q in range(MBUF):
                step(w + q, q, first=False)

        for wp in range(NWIN - GDIST, NWIN):
            scatter_wait(wp % RBUF, wp % MBUF)

        plsc.subcore_barrier()

        last_rows = N_NODES - (NS - 1) * ROWS_PER_SUBCORE
        col0 = pl.multiple_of(c * HALF, HALF)

        @pl.when(s < NS - 1)
        def _():
            pltpu.sync_copy(
                acc.at[pl.ds(row0, ROWS_PER_SUBCORE)],
                out_hbm.at[pl.ds(row0, ROWS_PER_SUBCORE), pl.ds(col0, HALF)])

        @pl.when(s == NS - 1)
        def _():
            pltpu.sync_copy(
                acc.at[pl.ds(row0, last_rows)],
                out_hbm.at[pl.ds(row0, last_rows), pl.ds(col0, HALF)])

    return sc_kernel(meta3, hab, b2)


def _tc_project_body(h_ref, w_ref, o_ref):
    dn = (((1,), (1,)), ((), ()))
    h_blk = h_ref[...]
    o_ref[0] = lax.dot_general(h_blk, w_ref[0:HALF, :], dn,
                               preferred_element_type=jnp.float32)
    o_ref[1] = lax.dot_general(h_blk, w_ref[HALF:D_OUT, :], dn,
                               preferred_element_type=jnp.float32)


def _tc_project(h, W):
    blk = 1000
    grid = (N_NODES // blk,)
    return pl.pallas_call(
        _tc_project_body,
        grid=grid,
        in_specs=[
            pl.BlockSpec((blk, D_IN), lambda i: (i, 0)),
            pl.BlockSpec((D_OUT, D_IN), lambda i: (0, 0)),
        ],
        out_specs=pl.BlockSpec((2, blk, HALF), lambda i: (0, i, 0)),
        out_shape=jax.ShapeDtypeStruct((2, N_NODES, HALF), jnp.float32),
    )(h, W)


def kernel(h, edge_index, edge_norm, W, b):
    src = edge_index[0].astype(jnp.int32)
    dst = edge_index[1].astype(jnp.int32)
    norm = edge_norm.reshape(-1).astype(jnp.float32)

    pad = E_PAD - N_EDGES
    fill = (jnp.arange(pad, dtype=jnp.int32) * 7919) % N_NODES
    src_p = jnp.concatenate([src, fill])
    dst_p = jnp.concatenate([dst, fill])
    norm_p = jnp.concatenate([norm, jnp.zeros((pad,), jnp.float32)])

    packed2 = (src_p | (dst_p << 16)).reshape(NS * NWIN, K)
    nbits2 = norm_p.view(jnp.int32).reshape(NS * NWIN, K)
    meta3 = jnp.stack([packed2, nbits2], axis=1)

    hab = _tc_project(h, W)
    return _sc_agg(meta3, hab, b.reshape(NC, HALF))
